# Initial kernel scaffold; baseline (speedup 1.0000x reference)
import jax
import jax.numpy as jnp
from jax.experimental import pallas as pl


def kernel(x, codebooks, sample_rate):
    raise NotImplementedError("write your pallas kernel here")


# speedup vs baseline: 1.3741x; 1.3741x over previous
import math
import functools

import jax
import jax.numpy as jnp
from jax import lax
from jax.experimental import pallas as pl
from jax.experimental.pallas import tpu as pltpu

N_Q = 8
BINS = 1024
DIM = 256
TB = 512


def _rvq_kernel(x_ref, cb_ref, cbt_ref, c2_ref, quant_ref, codes_ref, ssq_ref):
    first = (pl.program_id(0) == 0) & (pl.program_id(1) == 0)

    @pl.when(first)
    def _():
        ssq_ref[...] = jnp.zeros_like(ssq_ref)

    rT = x_ref[0]
    xT = rT
    iota0 = lax.broadcasted_iota(jnp.int32, (BINS, TB), 0)
    for i in range(N_Q):
        dotsT = jnp.dot(cb_ref[i], rT, preferred_element_type=jnp.float32)
        r2 = jnp.sum(rT * rT, axis=0, keepdims=True)
        distT = r2 - 2.0 * dotsT + c2_ref[i]
        m = jnp.min(distT, axis=0, keepdims=True)
        idx_row = jnp.min(jnp.where(distT == m, iota0, BINS), axis=0, keepdims=True)
        codes_ref[0, i] = idx_row[0]
        onehotT = (iota0 == idx_row).astype(jnp.float32)
        qT = jnp.dot(cbt_ref[i], onehotT, preferred_element_type=jnp.float32,
                     precision=lax.Precision.HIGHEST)
        rT = rT - qT
        ssq_ref[:, i : i + 1] += jnp.sum(rT * rT, axis=1, keepdims=True)
    quant_ref[0] = xT - rT


@functools.partial(jax.jit, static_argnames=())
def kernel(x, codebooks, sample_rate):
    B, D, T = x.shape
    cbT = jnp.transpose(codebooks, (0, 2, 1))
    c2 = jnp.sum(codebooks * codebooks, axis=-1)[..., None]

    grid = (B, T // TB)
    quantized, codes_t, ssq = pl.pallas_call(
        _rvq_kernel,
        grid=grid,
        in_specs=[
            pl.BlockSpec((1, DIM, TB), lambda b, t: (b, 0, t)),
            pl.BlockSpec((N_Q, BINS, DIM), lambda b, t: (0, 0, 0)),
            pl.BlockSpec((N_Q, DIM, BINS), lambda b, t: (0, 0, 0)),
            pl.BlockSpec((N_Q, BINS, 1), lambda b, t: (0, 0, 0)),
        ],
        out_specs=[
            pl.BlockSpec((1, DIM, TB), lambda b, t: (b, 0, t)),
            pl.BlockSpec((1, N_Q, TB), lambda b, t: (b, 0, t)),
            pl.BlockSpec((DIM, N_Q), lambda b, t: (0, 0)),
        ],
        out_shape=[
            jax.ShapeDtypeStruct((B, D, T), jnp.float32),
            jax.ShapeDtypeStruct((B, N_Q, T), jnp.int32),
            jax.ShapeDtypeStruct((DIM, N_Q), jnp.float32),
        ],
    )(x, codebooks, cbT, c2)

    codes = jnp.transpose(codes_t, (1, 0, 2))
    losses = 1.25 * jnp.sum(ssq, axis=0) / (B * T * D)
    commit_loss = jnp.mean(losses)
    bw = jnp.asarray(N_Q * math.log2(BINS) * sample_rate / 1000.0, dtype=x.dtype)
    return quantized, codes, bw, commit_loss


# speedup vs baseline: 2.1519x; 1.5661x over previous
import math
import functools

import jax
import jax.numpy as jnp
from jax import lax
from jax.experimental import pallas as pl
from jax.experimental.pallas import tpu as pltpu

N_Q = 8
BINS = 1024
DIM = 256
TB = 512


def _rvq_kernel(x_ref, cb_ref, cbthi_ref, cbtmid_ref, cbtlo_ref, c2_ref,
                quant_ref, codes_ref, ssq_ref):
    first = (pl.program_id(0) == 0) & (pl.program_id(1) == 0)

    @pl.when(first)
    def _():
        ssq_ref[...] = jnp.zeros_like(ssq_ref)

    rT = x_ref[0]
    xT = rT
    iota0 = lax.broadcasted_iota(jnp.int32, (BINS, TB), 0)
    for i in range(N_Q):
        dotsT = jnp.dot(cb_ref[i], rT, preferred_element_type=jnp.float32)
        r2 = jnp.sum(rT * rT, axis=0, keepdims=True)
        distT = r2 - 2.0 * dotsT + c2_ref[i]
        m = jnp.min(distT, axis=0, keepdims=True)
        idx_row = jnp.min(jnp.where(distT == m, iota0, BINS), axis=0, keepdims=True)
        codes_ref[0, i] = idx_row[0]
        onehotT = (iota0 == idx_row).astype(jnp.bfloat16)
        qT = (jnp.dot(cbthi_ref[i], onehotT, preferred_element_type=jnp.float32)
              + jnp.dot(cbtmid_ref[i], onehotT, preferred_element_type=jnp.float32)
              + jnp.dot(cbtlo_ref[i], onehotT, preferred_element_type=jnp.float32))
        rT = rT - qT
        ssq_ref[:, i : i + 1] += jnp.sum(rT * rT, axis=1, keepdims=True)
    quant_ref[0] = xT - rT


@functools.partial(jax.jit, static_argnames=())
def kernel(x, codebooks, sample_rate):
    B, D, T = x.shape
    cbT = jnp.transpose(codebooks, (0, 2, 1))
    cbt_hi = cbT.astype(jnp.bfloat16)
    rem1 = cbT - cbt_hi.astype(jnp.float32)
    cbt_mid = rem1.astype(jnp.bfloat16)
    cbt_lo = (rem1 - cbt_mid.astype(jnp.float32)).astype(jnp.bfloat16)
    c2 = jnp.sum(codebooks * codebooks, axis=-1)[..., None]

    grid = (B, T // TB)
    quantized, codes_t, ssq = pl.pallas_call(
        _rvq_kernel,
        grid=grid,
        in_specs=[
            pl.BlockSpec((1, DIM, TB), lambda b, t: (b, 0, t)),
            pl.BlockSpec((N_Q, BINS, DIM), lambda b, t: (0, 0, 0)),
            pl.BlockSpec((N_Q, DIM, BINS), lambda b, t: (0, 0, 0)),
            pl.BlockSpec((N_Q, DIM, BINS), lambda b, t: (0, 0, 0)),
            pl.BlockSpec((N_Q, DIM, BINS), lambda b, t: (0, 0, 0)),
            pl.BlockSpec((N_Q, BINS, 1), lambda b, t: (0, 0, 0)),
        ],
        out_specs=[
            pl.BlockSpec((1, DIM, TB), lambda b, t: (b, 0, t)),
            pl.BlockSpec((1, N_Q, TB), lambda b, t: (b, 0, t)),
            pl.BlockSpec((DIM, N_Q), lambda b, t: (0, 0)),
        ],
        out_shape=[
            jax.ShapeDtypeStruct((B, D, T), jnp.float32),
            jax.ShapeDtypeStruct((B, N_Q, T), jnp.int32),
            jax.ShapeDtypeStruct((DIM, N_Q), jnp.float32),
        ],
    )(x, codebooks, cbt_hi, cbt_mid, cbt_lo, c2)

    codes = jnp.transpose(codes_t, (1, 0, 2))
    losses = 1.25 * jnp.sum(ssq, axis=0) / (B * T * D)
    commit_loss = jnp.mean(losses)
    bw = jnp.asarray(N_Q * math.log2(BINS) * sample_rate / 1000.0, dtype=x.dtype)
    return quantized, codes, bw, commit_loss


# speedup vs baseline: 2.1831x; 1.0145x over previous
import math
import functools

import jax
import jax.numpy as jnp
from jax import lax
from jax.experimental import pallas as pl
from jax.experimental.pallas import tpu as pltpu

N_Q = 8
BINS = 1024
DIM = 256
TB = 512


def _rvq_kernel(x_ref, cb_ref, cbthi_ref, cbtmid_ref, cbtlo_ref, c2_ref,
                quant_ref, codes_ref, ssq_ref):
    first = (pl.program_id(0) == 0) & (pl.program_id(1) == 0)

    @pl.when(first)
    def _():
        ssq_ref[...] = jnp.zeros_like(ssq_ref)

    rT = x_ref[0]
    xT = rT
    iota0 = lax.broadcasted_iota(jnp.int32, (BINS, TB), 0)
    for i in range(N_Q):
        dotsT = jnp.dot(cb_ref[i], rT, preferred_element_type=jnp.float32)
        r2 = jnp.sum(rT * rT, axis=0, keepdims=True)
        distT = r2 - 2.0 * dotsT + c2_ref[i]
        m = jnp.min(distT, axis=0, keepdims=True)
        idx_row = jnp.min(jnp.where(distT == m, iota0, BINS), axis=0, keepdims=True)
        codes_ref[0, i] = idx_row[0]
        onehotT = (iota0 == idx_row).astype(jnp.bfloat16)
        qT = (jnp.dot(cbthi_ref[i], onehotT, preferred_element_type=jnp.float32)
              + jnp.dot(cbtmid_ref[i], onehotT, preferred_element_type=jnp.float32)
              + jnp.dot(cbtlo_ref[i], onehotT, preferred_element_type=jnp.float32))
        d = qT - rT
        qst = rT + d
        rT = rT - qst
        ssq_ref[:, i : i + 1] += jnp.sum(d * d, axis=1, keepdims=True)
    quant_ref[0] = xT - rT


@functools.partial(jax.jit, static_argnames=())
def kernel(x, codebooks, sample_rate):
    B, D, T = x.shape
    cbT = jnp.transpose(codebooks, (0, 2, 1))
    top16 = jnp.uint32(0xFFFF0000)
    u = lax.bitcast_convert_type(cbT, jnp.uint32)
    h1 = lax.bitcast_convert_type(u & top16, jnp.float32)
    r1 = cbT - h1
    h2 = lax.bitcast_convert_type(lax.bitcast_convert_type(r1, jnp.uint32) & top16,
                                  jnp.float32)
    r2 = r1 - h2
    cbt_hi = h1.astype(jnp.bfloat16)
    cbt_mid = h2.astype(jnp.bfloat16)
    cbt_lo = r2.astype(jnp.bfloat16)
    c2 = jnp.sum(codebooks * codebooks, axis=-1)[..., None]

    grid = (B, T // TB)
    quantized, codes_t, ssq = pl.pallas_call(
        _rvq_kernel,
        grid=grid,
        in_specs=[
            pl.BlockSpec((1, DIM, TB), lambda b, t: (b, 0, t)),
            pl.BlockSpec((N_Q, BINS, DIM), lambda b, t: (0, 0, 0)),
            pl.BlockSpec((N_Q, DIM, BINS), lambda b, t: (0, 0, 0)),
            pl.BlockSpec((N_Q, DIM, BINS), lambda b, t: (0, 0, 0)),
            pl.BlockSpec((N_Q, DIM, BINS), lambda b, t: (0, 0, 0)),
            pl.BlockSpec((N_Q, BINS, 1), lambda b, t: (0, 0, 0)),
        ],
        out_specs=[
            pl.BlockSpec((1, DIM, TB), lambda b, t: (b, 0, t)),
            pl.BlockSpec((1, N_Q, TB), lambda b, t: (b, 0, t)),
            pl.BlockSpec((DIM, N_Q), lambda b, t: (0, 0)),
        ],
        out_shape=[
            jax.ShapeDtypeStruct((B, D, T), jnp.float32),
            jax.ShapeDtypeStruct((B, N_Q, T), jnp.int32),
            jax.ShapeDtypeStruct((DIM, N_Q), jnp.float32),
        ],
    )(x, codebooks, cbt_hi, cbt_mid, cbt_lo, c2)

    codes = jnp.transpose(codes_t, (1, 0, 2))
    losses = 1.25 * jnp.sum(ssq, axis=0) / (B * T * D)
    commit_loss = jnp.mean(losses)
    bw = jnp.asarray(N_Q * math.log2(BINS) * sample_rate / 1000.0, dtype=x.dtype)
    return quantized, codes, bw, commit_loss
